# baseline (speedup 1.0000x reference)
import functools

import jax
import jax.numpy as jnp
from jax import lax
from jax.experimental import pallas as pl
from jax.experimental.pallas import tpu as pltpu
from jax.experimental.pallas import tpu_sc as plsc

_NC = 2
_NS = 16
_NW = _NC * _NS
_L = 16

_CH = 32
_NBUF = 3


@functools.lru_cache(maxsize=None)
def _sc_scores_fn(b, s, d):
    n_tok = b * s
    tok_per_w = n_tok // _NW
    nch = tok_per_w // _CH
    wpr = s // tok_per_w
    mesh = plsc.VectorSubcoreMesh(core_axis_name="c", subcore_axis_name="s")

    def body(ids_hbm, table_hbm, out_hbm, idx_v, rows0, rows1, rows2, sc_v,
             sem0, sem1, sem2):
        wid = lax.axis_index("s") * _NC + lax.axis_index("c")
        wrow = wid // wpr
        wcol = (wid % wpr) * tok_per_w
        pltpu.sync_copy(ids_hbm.at[wrow, pl.ds(wcol, tok_per_w)], idx_v)

        bufs = (rows0, rows1, rows2)
        sems = (sem0, sem1, sem2)

        def start(c):
            return pltpu.async_copy(
                table_hbm.at[idx_v.at[pl.ds(c * _CH, _CH)]],
                bufs[c % _NBUF], sems[c % _NBUF])

        lane15 = lax.iota(jnp.int32, _L) == (_L - 1)

        def compute(c, prev):
            buf = bufs[c % _NBUF]

            def tbody(t, prev, buf=buf, c=c):
                cs = plsc.cumsum(prev)
                g = c * _CH + t
                m = lane15 if c else lane15 & (t > 0)
                plsc.store_scatter(sc_v, [jnp.full((_L,), g - 1, jnp.int32)],
                                   cs, mask=m)
                row = buf.at[t]
                accs = [jnp.zeros((_L,), jnp.float32) for _ in range(4)]
                for j in range(d // _L):
                    x = row[pl.ds(j * _L, _L)]
                    accs[j % 4] = accs[j % 4] + x * x
                return (accs[0] + accs[1]) + (accs[2] + accs[3])

            return lax.fori_loop(0, _CH, tbody, prev)

        cur = [start(0), start(1)]
        prev = jnp.zeros((_L,), jnp.float32)
        for c in range(nch):
            if c + 2 < nch:
                cur.append(start(c + 2))
            cur[c].wait()
            prev = compute(c, prev)
        cs = plsc.cumsum(prev)
        plsc.store_scatter(sc_v, [jnp.full((_L,), tok_per_w - 1, jnp.int32)],
                           cs, mask=lane15)
        pltpu.sync_copy(sc_v, out_hbm.at[wrow, pl.ds(wcol, tok_per_w)])

    return pl.kernel(
        body,
        mesh=mesh,
        compiler_params=pltpu.CompilerParams(
            use_tc_tiling_on_sc=False, needs_layout_passes=False),
        out_type=jax.ShapeDtypeStruct((b, s), jnp.float32),
        scratch_types=[
            pltpu.VMEM((tok_per_w,), jnp.int32),
            pltpu.VMEM((_CH, d), jnp.float32),
            pltpu.VMEM((_CH, d), jnp.float32),
            pltpu.VMEM((_CH, d), jnp.float32),
            pltpu.VMEM((tok_per_w,), jnp.float32),
            pltpu.SemaphoreType.DMA,
            pltpu.SemaphoreType.DMA,
            pltpu.SemaphoreType.DMA,
        ],
    )


def _mask_body(k, b, s, scores_ref, out_ref):
    sc = jnp.sqrt(scores_ref[...])
    bits = lax.bitcast_convert_type(sc, jnp.int32)
    idx = lax.broadcasted_iota(jnp.int32, (b, s), 1)
    kk = jnp.int32(k)

    def tbody(i, lo):
        t = lo + jnp.left_shift(jnp.int32(1), jnp.int32(30) - i)
        cnt = jnp.sum((bits >= t).astype(jnp.int32), axis=1, keepdims=True)
        return jnp.where(cnt >= kk, t, lo)

    thr = lax.fori_loop(0, 31, tbody, jnp.zeros((b, 1), jnp.int32))

    gt = bits > thr
    tie = bits == thr
    need = kk - jnp.sum(gt.astype(jnp.int32), axis=1, keepdims=True)

    nbits = max(1, (s - 1).bit_length())

    def cbody(i, c):
        cand = c + jnp.left_shift(jnp.int32(1), jnp.int32(nbits - 1) - i)
        cnt = jnp.sum((tie & (idx < cand)).astype(jnp.int32), axis=1,
                      keepdims=True)
        return jnp.where(cnt < need, cand, c)

    cut = lax.fori_loop(0, nbits, cbody, jnp.zeros((b, 1), jnp.int32))

    out_ref[...] = (gt | (tie & (idx <= cut))).astype(jnp.float32)


def kernel(input_ids, emb_weight):
    b, s = input_ids.shape
    _, d = emb_weight.shape
    k = int(s * 0.9)
    scores = _sc_scores_fn(b, s, d)(input_ids.astype(jnp.int32), emb_weight)
    return pl.pallas_call(
        functools.partial(_mask_body, k, b, s),
        out_shape=jax.ShapeDtypeStruct((b, s), jnp.float32),
    )(scores)
